# baseline (speedup 1.0000x reference)
import functools

import jax
import jax.numpy as jnp
from jax import lax
from jax.experimental import pallas as pl
from jax.experimental.pallas import tpu as pltpu
from jax.experimental.pallas import tpu_sc as plsc

NUM_T = 1000
IMG_SHAPE = (3, 64, 64)
BATCH = 1024
FEAT = 3 * 64 * 64

_NC = 2
_NS = 16
_L = 16
_NW = _NC * _NS
_PER_W = BATCH // _NW
_TBL_PAD = 1024


def _schedule_tables():
    scale = 1000.0 / NUM_T
    beta = jnp.linspace(scale * 0.0001, scale * 0.02, NUM_T, dtype=jnp.float32)
    alpha_cum = jnp.cumprod(1.0 - beta, axis=0)
    sqrt_ac = jnp.sqrt(alpha_cum)
    sqrt_omac = jnp.sqrt(1.0 - alpha_cum)
    pad = _TBL_PAD - NUM_T
    return (jnp.pad(sqrt_ac, (0, pad)), jnp.pad(sqrt_omac, (0, pad)))


def _sc_gather_body(ts_hbm, ac_hbm, om_hbm, coef_hbm, std_hbm,
                    ac_v, om_v, idx_v, coef_v, std_v):
    wid = lax.axis_index("s") * _NC + lax.axis_index("c")
    base = wid * _PER_W
    pltpu.sync_copy(ac_hbm, ac_v)
    pltpu.sync_copy(om_hbm, om_v)
    pltpu.sync_copy(ts_hbm.at[pl.ds(base, _PER_W)], idx_v)
    for j in range(_PER_W // _L):
        idx = idx_v[pl.ds(j * _L, _L)]
        coef_v[pl.ds(j * _L, _L)] = plsc.load_gather(ac_v, [idx])
        std_v[pl.ds(j * _L, _L)] = plsc.load_gather(om_v, [idx])
    pltpu.sync_copy(coef_v, coef_hbm.at[pl.ds(base, _PER_W)])
    pltpu.sync_copy(std_v, std_hbm.at[pl.ds(base, _PER_W)])


_sc_gather = pl.kernel(
    _sc_gather_body,
    out_type=(
        jax.ShapeDtypeStruct((BATCH,), jnp.float32),
        jax.ShapeDtypeStruct((BATCH,), jnp.float32),
    ),
    mesh=plsc.VectorSubcoreMesh(core_axis_name="c", subcore_axis_name="s"),
    compiler_params=pltpu.CompilerParams(needs_layout_passes=False),
    scratch_types=[
        pltpu.VMEM((_TBL_PAD,), jnp.float32),
        pltpu.VMEM((_TBL_PAD,), jnp.float32),
        pltpu.VMEM((_PER_W,), jnp.int32),
        pltpu.VMEM((_PER_W,), jnp.float32),
        pltpu.VMEM((_PER_W,), jnp.float32),
    ],
)


_ROWS = 64


def _scale_body(coef_ref, std_ref, x0_ref, eps_ref, out_ref):
    out_ref[...] = coef_ref[...] * x0_ref[...] + std_ref[...] * eps_ref[...]


@functools.partial(jax.jit, static_argnames=())
def _tc_scale(coef, std, x2, e2):
    grid = (BATCH // _ROWS,)
    row_spec = pl.BlockSpec((_ROWS, FEAT), lambda i: (i, 0))
    s_spec = pl.BlockSpec((_ROWS, 1), lambda i: (i, 0))
    return pl.pallas_call(
        _scale_body,
        grid=grid,
        in_specs=[s_spec, s_spec, row_spec, row_spec],
        out_specs=row_spec,
        out_shape=jax.ShapeDtypeStruct((BATCH, FEAT), jnp.float32),
    )(coef, std, x2, e2)


def kernel(x0, timesteps, eps):
    sqrt_ac, sqrt_omac = _schedule_tables()
    coef, std = _sc_gather(timesteps.astype(jnp.int32), sqrt_ac, sqrt_omac)
    x2 = x0.reshape(BATCH, FEAT)
    e2 = eps.reshape(BATCH, FEAT)
    sample = _tc_scale(coef.reshape(BATCH, 1), std.reshape(BATCH, 1), x2, e2)
    return (sample.reshape(x0.shape), eps)
